# baseline (speedup 1.0000x reference)
import functools

import jax
import jax.numpy as jnp
from jax import lax
from jax.experimental import pallas as pl
from jax.experimental.pallas import tpu as pltpu
from jax.experimental.pallas import tpu_sc as plsc

_NC = 2
_NS = 16
_NW = _NC * _NS
_L = 16
_RB = 40
_G = 8


@functools.partial(jax.jit, static_argnums=(2, 3))
def _lookup_call(table, data_t, n_rows, n_cols):
    cols_per_w = n_cols // _NW
    nblk = n_rows // _RB
    nslots = cols_per_w // _L
    mesh = plsc.VectorSubcoreMesh(core_axis_name="c", subcore_axis_name="s")

    @functools.partial(
        pl.kernel,
        mesh=mesh,
        out_type=jax.ShapeDtypeStruct((n_rows, n_cols), jnp.int32),
        scratch_types=[
            pltpu.VMEM((1000,), jnp.int32),
            pltpu.VMEM((_RB, cols_per_w), jnp.int32),
            pltpu.VMEM((_RB, cols_per_w), jnp.int32),
            pltpu.VMEM((_RB, cols_per_w), jnp.int32),
            pltpu.VMEM((_RB, cols_per_w), jnp.int32),
            pltpu.SemaphoreType.DMA,
            pltpu.SemaphoreType.DMA,
            pltpu.SemaphoreType.DMA,
            pltpu.SemaphoreType.DMA,
        ],
        compiler_params=pltpu.CompilerParams(needs_layout_passes=False),
    )
    def lookup(table_hbm, data_hbm, out_hbm, table_v, idx_v0, idx_v1,
               res_v0, res_v1, in_sem0, in_sem1, out_sem0, out_sem1):
        wid = lax.axis_index("s") * _NC + lax.axis_index("c")
        col0 = wid * cols_per_w
        idx_bufs = (idx_v0, idx_v1)
        res_bufs = (res_v0, res_v1)
        in_sems = (in_sem0, in_sem1)
        out_sems = (out_sem0, out_sem1)
        pltpu.sync_copy(table_hbm, table_v)

        in_h = {}
        out_h = {}
        in_h[0] = pltpu.async_copy(
            data_hbm.at[pl.ds(0, _RB), pl.ds(col0, cols_per_w)],
            idx_bufs[0], in_sems[0])
        for b in range(nblk):
            s = b % 2
            if b + 1 < nblk:
                in_h[b + 1] = pltpu.async_copy(
                    data_hbm.at[pl.ds((b + 1) * _RB, _RB),
                                pl.ds(col0, cols_per_w)],
                    idx_bufs[(b + 1) % 2], in_sems[(b + 1) % 2])
            in_h[b].wait()
            if b >= 2:
                out_h[b - 2].wait()

            @pl.loop(0, _RB)
            def row_body(r):
                @pl.loop(0, nslots // _G)
                def grp_body(g):
                    c0 = g * (_G * _L)
                    ivs = [idx_bufs[s][r, pl.ds(c0 + k * _L, _L)]
                           for k in range(_G)]
                    tvs = [plsc.load_gather(table_v, [iv]) for iv in ivs]
                    for k, tv in enumerate(tvs):
                        res_bufs[s][r, pl.ds(c0 + k * _L, _L)] = tv

            out_h[b] = pltpu.async_copy(
                res_bufs[s],
                out_hbm.at[pl.ds(b * _RB, _RB), pl.ds(col0, cols_per_w)],
                out_sems[s])
        for b in range(max(nblk - 2, 0), nblk):
            out_h[b].wait()

    return lookup(table, data_t)


def kernel(data, table):
    out_t = _lookup_call(table, data.T, data.shape[1], data.shape[0])
    return out_t.T


# speedup vs baseline: 1.1073x; 1.1073x over previous
import functools

import jax
import jax.numpy as jnp
from jax import lax
from jax.experimental import pallas as pl
from jax.experimental.pallas import tpu as pltpu
from jax.experimental.pallas import tpu_sc as plsc

_NC = 2
_NS = 16
_NW = _NC * _NS
_L = 16
_RB = 40


@functools.partial(jax.jit, static_argnums=(2, 3))
def _lookup_call(table, data_t, n_rows, n_cols):
    cols_per_w = n_cols // _NW
    nblk = n_rows // _RB
    nslots = cols_per_w // _L
    mesh = plsc.VectorSubcoreMesh(core_axis_name="c", subcore_axis_name="s")

    @functools.partial(
        pl.kernel,
        mesh=mesh,
        out_type=jax.ShapeDtypeStruct((n_rows, n_cols), jnp.int32),
        scratch_types=[
            pltpu.VMEM((1000,), jnp.int32),
            pltpu.VMEM((_RB, cols_per_w), jnp.int32),
            pltpu.VMEM((_RB, cols_per_w), jnp.int32),
            pltpu.VMEM((_RB, cols_per_w), jnp.int32),
            pltpu.VMEM((_RB, cols_per_w), jnp.int32),
            pltpu.SemaphoreType.DMA,
            pltpu.SemaphoreType.DMA,
            pltpu.SemaphoreType.DMA,
            pltpu.SemaphoreType.DMA,
        ],
        compiler_params=pltpu.CompilerParams(
            needs_layout_passes=False,
            skip_device_barrier=True,
            disable_bounds_checks=True,
            disable_semaphore_checks=True,
        ),
    )
    def lookup(table_hbm, data_hbm, out_hbm, table_v, idx_v0, idx_v1,
               res_v0, res_v1, in_sem0, in_sem1, out_sem0, out_sem1):
        wid = lax.axis_index("s") * _NC + lax.axis_index("c")
        col0 = wid * cols_per_w
        idx_bufs = (idx_v0, idx_v1)
        res_bufs = (res_v0, res_v1)
        in_sems = (in_sem0, in_sem1)
        out_sems = (out_sem0, out_sem1)
        pltpu.sync_copy(table_hbm, table_v)

        in_h = {}
        out_h = {}
        in_h[0] = pltpu.async_copy(
            data_hbm.at[pl.ds(0, _RB), pl.ds(col0, cols_per_w)],
            idx_bufs[0], in_sems[0])
        for b in range(nblk):
            s = b % 2
            if b + 1 < nblk:
                in_h[b + 1] = pltpu.async_copy(
                    data_hbm.at[pl.ds((b + 1) * _RB, _RB),
                                pl.ds(col0, cols_per_w)],
                    idx_bufs[(b + 1) % 2], in_sems[(b + 1) % 2])
            in_h[b].wait()
            if b >= 2:
                out_h[b - 2].wait()

            @pl.loop(0, _RB)
            def row_body(r):
                ivs = [idx_bufs[s][r, pl.ds(c * _L, _L)]
                       for c in range(nslots)]
                tvs = [plsc.load_gather(table_v, [iv]) for iv in ivs]
                for c, tv in enumerate(tvs):
                    res_bufs[s][r, pl.ds(c * _L, _L)] = tv

            out_h[b] = pltpu.async_copy(
                res_bufs[s],
                out_hbm.at[pl.ds(b * _RB, _RB), pl.ds(col0, cols_per_w)],
                out_sems[s])
        for b in range(max(nblk - 2, 0), nblk):
            out_h[b].wait()

    return lookup(table, data_t)


def kernel(data, table):
    out_t = _lookup_call(table, data.T, data.shape[1], data.shape[0])
    return out_t.T
